# baseline (speedup 1.0000x reference)
import functools

import jax
import jax.numpy as jnp
from jax.experimental import pallas as pl

_DN = ('NCHW', 'OIHW', 'NCHW')


def _conv2d(x, w, b, stride, pad):
    y = jax.lax.conv_general_dilated(
        x, w, (stride, stride), [(pad, pad), (pad, pad)], dimension_numbers=_DN)
    return y + b[None, :, None, None]


def _convT2d(x, w, b, stride, pad, out_pad):
    w_ = jnp.flip(w, axis=(2, 3)).transpose(1, 0, 2, 3)
    k = w.shape[2]
    p = k - 1 - pad
    y = jax.lax.conv_general_dilated(
        x, w_, (1, 1), [(p, p + out_pad), (p, p + out_pad)],
        lhs_dilation=(stride, stride), dimension_numbers=_DN)
    return y + b[None, :, None, None]


def _vq_body(f_ref, cb_ref, fn_ref, cbn_ref, idx_ref, q_ref, loss_ref, *,
             blk, K):
    f = f_ref[...]
    cb = cb_ref[...]
    fn = fn_ref[...]
    cbn = cbn_ref[...]
    scores = jax.lax.dot_general(
        f, cb, dimension_numbers=(((1,), (1,)), ((), ())),
        preferred_element_type=jnp.float32)
    d = fn + cbn - 2.0 * scores
    dmin = jnp.min(d, axis=1, keepdims=True)
    kiota = jax.lax.broadcasted_iota(jnp.int32, (blk, K), 1)
    idx = jnp.min(jnp.where(d == dmin, kiota, K), axis=1)
    idx_ref[...] = idx[:, None].astype(jnp.int32)
    onehot = (kiota == idx[:, None]).astype(jnp.float32)
    q_ref[...] = jax.lax.dot_general(
        onehot, cb, dimension_numbers=(((1,), (0,)), ((), ())),
        preferred_element_type=jnp.float32)
    part = jnp.sum(dmin, axis=0, keepdims=True)

    @pl.when(pl.program_id(0) == 0)
    def _init():
        loss_ref[...] = jnp.zeros_like(part)

    loss_ref[...] += part


def _vq(flat, codebook, *, blk=256):
    n, d = flat.shape
    k = codebook.shape[0]
    grid = n // blk
    fn = jnp.sum(flat ** 2, axis=1, keepdims=True)
    cbn = jnp.sum(codebook ** 2, axis=1)[None, :]
    idx, q, loss = pl.pallas_call(
        functools.partial(_vq_body, blk=blk, K=k),
        grid=(grid,),
        in_specs=[
            pl.BlockSpec((blk, d), lambda i: (i, 0)),
            pl.BlockSpec((k, d), lambda i: (0, 0)),
            pl.BlockSpec((blk, 1), lambda i: (i, 0)),
            pl.BlockSpec((1, k), lambda i: (0, 0)),
        ],
        out_specs=[
            pl.BlockSpec((blk, 1), lambda i: (i, 0)),
            pl.BlockSpec((blk, d), lambda i: (i, 0)),
            pl.BlockSpec((1, 1), lambda i: (0, 0)),
        ],
        out_shape=[
            jax.ShapeDtypeStruct((n, 1), jnp.int32),
            jax.ShapeDtypeStruct((n, d), jnp.float32),
            jax.ShapeDtypeStruct((1, 1), jnp.float32),
        ],
    )(flat, codebook, fn, cbn)
    return idx, q, loss[0, 0]


def kernel(x, ew1, eb1, ew2, eb2, ew3, eb3, codebook, dw1, db1, dw2, db2,
           dw3, db3):
    commitment_cost = 0.25
    z = jax.nn.relu(_conv2d(x, ew1, eb1, 2, 1))
    z = jax.nn.relu(_conv2d(z, ew2, eb2, 2, 1))
    z = _conv2d(z, ew3, eb3, 2, 1)
    B, D, H, W = z.shape
    flat = z.transpose(0, 2, 3, 1).reshape(-1, D)

    idx, quantized, loss_sum = _vq(flat, codebook)
    vq_loss = (1.0 + commitment_cost) * loss_sum / (flat.shape[0] * D)

    z_q = quantized.reshape(B, H, W, D).transpose(0, 3, 1, 2)
    h = jax.nn.relu(_convT2d(z_q, dw1, db1, 2, 1, 0))
    h = jax.nn.relu(_convT2d(h, dw2, db2, 2, 1, 0))
    x_recon = jax.nn.sigmoid(_convT2d(h, dw3, db3, 2, 0, 1))
    return (x_recon, vq_loss, idx)


# speedup vs baseline: 1.0000x; 1.0000x over previous
import functools

import jax
import jax.numpy as jnp
from jax.experimental import pallas as pl

_DN = ('NCHW', 'OIHW', 'NCHW')


def _conv2d(x, w, b, stride, pad):
    y = jax.lax.conv_general_dilated(
        x, w, (stride, stride), [(pad, pad), (pad, pad)], dimension_numbers=_DN)
    return y + b[None, :, None, None]


def _convT2d(x, w, b, stride, pad, out_pad):
    w_ = jnp.flip(w, axis=(2, 3)).transpose(1, 0, 2, 3)
    k = w.shape[2]
    p = k - 1 - pad
    y = jax.lax.conv_general_dilated(
        x, w_, (1, 1), [(p, p + out_pad), (p, p + out_pad)],
        lhs_dilation=(stride, stride), dimension_numbers=_DN)
    return y + b[None, :, None, None]


def _vq_body(f_ref, cb_ref, fn_ref, cbn_ref, idx_ref, q_ref, loss_ref, *,
             blk, K):
    f = f_ref[...]
    cb = cb_ref[...]
    fn = fn_ref[...]
    cbn = cbn_ref[...]
    scores = jax.lax.dot_general(
        f, cb, dimension_numbers=(((1,), (1,)), ((), ())),
        preferred_element_type=jnp.float32)
    d = fn + cbn - 2.0 * scores
    dmin = jnp.min(d, axis=1, keepdims=True)
    kiota = jax.lax.broadcasted_iota(jnp.int32, (blk, K), 1)
    idx = jnp.min(jnp.where(d == dmin, kiota, K), axis=1)
    idx_ref[...] = idx[:, None].astype(jnp.int32)
    onehot = (kiota == idx[:, None]).astype(jnp.float32)
    q_ref[...] = jax.lax.dot_general(
        onehot, cb, dimension_numbers=(((1,), (0,)), ((), ())),
        preferred_element_type=jnp.float32)
    part = jnp.sum(dmin, axis=0, keepdims=True)

    @pl.when(pl.program_id(0) == 0)
    def _init():
        loss_ref[...] = jnp.zeros_like(part)

    loss_ref[...] += part


def _vq(flat, codebook, *, blk=256):
    n, d = flat.shape
    k = codebook.shape[0]
    grid = n // blk
    fn = jnp.sum(flat ** 2, axis=1, keepdims=True)
    cbn = jnp.sum(codebook ** 2, axis=1)[None, :]
    idx, q, loss = pl.pallas_call(
        functools.partial(_vq_body, blk=blk, K=k),
        grid=(grid,),
        in_specs=[
            pl.BlockSpec((blk, d), lambda i: (i, 0)),
            pl.BlockSpec((k, d), lambda i: (0, 0)),
            pl.BlockSpec((blk, 1), lambda i: (i, 0)),
            pl.BlockSpec((1, k), lambda i: (0, 0)),
        ],
        out_specs=[
            pl.BlockSpec((blk, 1), lambda i: (i, 0)),
            pl.BlockSpec((blk, d), lambda i: (i, 0)),
            pl.BlockSpec((1, 1), lambda i: (0, 0)),
        ],
        out_shape=[
            jax.ShapeDtypeStruct((n, 1), jnp.int32),
            jax.ShapeDtypeStruct((n, d), jnp.float32),
            jax.ShapeDtypeStruct((1, 1), jnp.float32),
        ],
    )(flat, codebook, fn, cbn)
    return idx, q, loss[0, 0]


_DNL = ('NHWC', 'HWIO', 'NHWC')


def _conv2d_nhwc(x, w, b, stride, pad):
    y = jax.lax.conv_general_dilated(
        x, w.transpose(2, 3, 1, 0), (stride, stride),
        [(pad, pad), (pad, pad)], dimension_numbers=_DNL)
    return y + b[None, None, None, :]


def _convT2d_nhwc(x, w, b, stride, pad, out_pad):
    w_ = jnp.flip(w, axis=(2, 3)).transpose(2, 3, 0, 1)
    k = w.shape[2]
    p = k - 1 - pad
    y = jax.lax.conv_general_dilated(
        x, w_, (1, 1), [(p, p + out_pad), (p, p + out_pad)],
        lhs_dilation=(stride, stride), dimension_numbers=_DNL)
    return y + b[None, None, None, :]


def kernel(x, ew1, eb1, ew2, eb2, ew3, eb3, codebook, dw1, db1, dw2, db2,
           dw3, db3):
    commitment_cost = 0.25
    xl = x.transpose(0, 2, 3, 1)
    z = jax.nn.relu(_conv2d_nhwc(xl, ew1, eb1, 2, 1))
    z = jax.nn.relu(_conv2d_nhwc(z, ew2, eb2, 2, 1))
    z = _conv2d_nhwc(z, ew3, eb3, 2, 1)
    B, H, W, D = z.shape
    flat = z.reshape(-1, D)

    idx, quantized, loss_sum = _vq(flat, codebook)
    vq_loss = (1.0 + commitment_cost) * loss_sum / (flat.shape[0] * D)

    z_q = quantized.reshape(B, H, W, D)
    h = jax.nn.relu(_convT2d_nhwc(z_q, dw1, db1, 2, 1, 0))
    h = jax.nn.relu(_convT2d_nhwc(h, dw2, db2, 2, 1, 0))
    x_recon = jax.nn.sigmoid(_convT2d_nhwc(h, dw3, db3, 2, 0, 1))
    return (x_recon.transpose(0, 3, 1, 2), vq_loss, idx)
